# baseline (speedup 1.0000x reference)
import jax
import jax.numpy as jnp
from jax.experimental import pallas as pl

_B, _S, _E = 4, 8192, 1024
_N = _B * _S
_R = 512


def _tc_body(num_ref, msk_ref, emb_ref, w_ref, b_ref, out_ref):
    n = num_ref[...]
    m = msk_ref[...]
    e = emb_ref[...]
    w = w_ref[...]
    bb = b_ref[...]
    fill = jnp.maximum(n * w + bb, 0.0)
    out_ref[...] = jnp.where(m > 0, fill, e)


def kernel(embeds, numbers, is_numbers, W, b):
    emb = embeds.reshape(_N, _E)
    num = numbers.reshape(_N, 1)
    msk = is_numbers.reshape(_N, 1).astype(jnp.int32)
    w2 = W.reshape(1, _E)
    b2 = b.reshape(1, _E)
    out = pl.pallas_call(
        _tc_body,
        grid=(_N // _R,),
        in_specs=[
            pl.BlockSpec((_R, 1), lambda i: (i, 0)),
            pl.BlockSpec((_R, 1), lambda i: (i, 0)),
            pl.BlockSpec((_R, _E), lambda i: (i, 0)),
            pl.BlockSpec((1, _E), lambda i: (0, 0)),
            pl.BlockSpec((1, _E), lambda i: (0, 0)),
        ],
        out_specs=pl.BlockSpec((_R, _E), lambda i: (i, 0)),
        out_shape=jax.ShapeDtypeStruct((_N, _E), jnp.float32),
    )(num, msk, emb, w2, b2)
    return out.reshape(_B, _S, _E)
